# baseline (speedup 1.0000x reference)
import jax
import jax.numpy as jnp
from jax.experimental import pallas as pl
from jax.experimental.pallas import tpu as pltpu

BLOCK_ROWS = 1024


def _copy_body(x_ref, o_ref, sizes_ref):
    @pl.when(pl.program_id(0) == 0)
    def _():
        sizes_ref[0] = jnp.int32(pl.num_programs(0) * BLOCK_ROWS)

    o_ref[...] = x_ref[...]


def kernel(x):
    n, d = x.shape
    gathered, sizes = pl.pallas_call(
        _copy_body,
        grid=(n // BLOCK_ROWS,),
        in_specs=[pl.BlockSpec((BLOCK_ROWS, d), lambda i: (i, 0))],
        out_specs=[
            pl.BlockSpec((BLOCK_ROWS, d), lambda i: (i, 0)),
            pl.BlockSpec(memory_space=pltpu.MemorySpace.SMEM),
        ],
        out_shape=[
            jax.ShapeDtypeStruct((n, d), x.dtype),
            jax.ShapeDtypeStruct((1,), jnp.int32),
        ],
    )(x)
    return (gathered, sizes)


# speedup vs baseline: 1.0204x; 1.0204x over previous
import jax
import jax.numpy as jnp
from jax.experimental import pallas as pl
from jax.experimental.pallas import tpu as pltpu

BLOCK_ROWS = 2048


def _copy_body(x_ref, o_ref, sizes_ref):
    @pl.when(pl.program_id(0) == 0)
    def _():
        sizes_ref[0] = jnp.int32(pl.num_programs(0) * BLOCK_ROWS)

    o_ref[...] = x_ref[...]


def kernel(x):
    n, d = x.shape
    gathered, sizes = pl.pallas_call(
        _copy_body,
        grid=(n // BLOCK_ROWS,),
        in_specs=[pl.BlockSpec((BLOCK_ROWS, d), lambda i: (i, 0))],
        out_specs=[
            pl.BlockSpec((BLOCK_ROWS, d), lambda i: (i, 0)),
            pl.BlockSpec(memory_space=pltpu.MemorySpace.SMEM),
        ],
        out_shape=[
            jax.ShapeDtypeStruct((n, d), x.dtype),
            jax.ShapeDtypeStruct((1,), jnp.int32),
        ],
    )(x)
    return (gathered, sizes)
